# Initial kernel scaffold; baseline (speedup 1.0000x reference)
import jax
import jax.numpy as jnp
from jax.experimental import pallas as pl


def kernel(timestamps, year_table, month_table, day_table, hour_table, proj_w, proj_b):
    raise NotImplementedError("write your pallas kernel here")


# speedup vs baseline: 9.3676x; 9.3676x over previous
import functools
import jax
import jax.numpy as jnp
from jax import lax
from jax.experimental import pallas as pl
from jax.experimental.pallas import tpu as pltpu

EMBED_DIM = 128
SUB = 32
OFF_Y, OFF_M, OFF_D, OFF_H = 0, 50, 62, 93
TOTAL_ROWS = 117

BLOCK_B = 2048


def _body(ts_ref, tpad_ref, pw_ref, pb_ref, out_ref):
    comb = jnp.dot(tpad_ref[...], pw_ref[...].T, preferred_element_type=jnp.float32)

    idx = ts_ref[...]
    cols = lax.broadcasted_iota(jnp.int32, (idx.shape[0], EMBED_DIM), 1)
    hot = (
        (cols == idx[:, 0:1] + OFF_Y)
        | (cols == idx[:, 1:2] + OFF_M)
        | (cols == idx[:, 2:3] + OFF_D)
        | (cols == idx[:, 3:4] + OFF_H)
    ).astype(jnp.float32)
    out_ref[...] = (
        jnp.dot(hot, comb, preferred_element_type=jnp.float32) + pb_ref[...]
    )


def kernel(timestamps, year_table, month_table, day_table, hour_table, proj_w, proj_b):
    B = timestamps.shape[0]
    ts = timestamps.astype(jnp.int32)

    tpad = jnp.zeros((EMBED_DIM, EMBED_DIM), dtype=jnp.float32)
    tpad = tpad.at[OFF_Y : OFF_Y + 50, 0 * SUB : 1 * SUB].set(year_table)
    tpad = tpad.at[OFF_M : OFF_M + 12, 1 * SUB : 2 * SUB].set(month_table)
    tpad = tpad.at[OFF_D : OFF_D + 31, 2 * SUB : 3 * SUB].set(day_table)
    tpad = tpad.at[OFF_H : OFF_H + 24, 3 * SUB : 4 * SUB].set(hour_table)

    grid = (B // BLOCK_B,)
    return pl.pallas_call(
        _body,
        grid=grid,
        in_specs=[
            pl.BlockSpec((BLOCK_B, 4), lambda i: (i, 0)),
            pl.BlockSpec((EMBED_DIM, EMBED_DIM), lambda i: (0, 0)),
            pl.BlockSpec((EMBED_DIM, EMBED_DIM), lambda i: (0, 0)),
            pl.BlockSpec((1, EMBED_DIM), lambda i: (0, 0)),
        ],
        out_specs=pl.BlockSpec((BLOCK_B, EMBED_DIM), lambda i: (i, 0)),
        out_shape=jax.ShapeDtypeStruct((B, EMBED_DIM), jnp.float32),
    )(ts, tpad, proj_w, proj_b.reshape(1, EMBED_DIM))
